# baseline (speedup 1.0000x reference)
import functools

import jax
import jax.numpy as jnp
from jax import lax
from jax.experimental import pallas as pl
from jax.experimental.pallas import tpu as pltpu
from jax.experimental.pallas import tpu_sc as plsc

_PROB = 0.12


def _neighbor_indices(T):
    k1, k2 = jax.random.split(jax.random.key(42))
    replace = jax.random.bernoulli(k1, _PROB, (T,))
    direction = jnp.where(jax.random.bernoulli(k2, 0.5, (T,)), 1, -1)
    idx = jnp.arange(T)
    direction = jnp.where(idx == 0, 1, direction)
    direction = jnp.where(idx == T - 1, -1, direction)
    return jnp.where(replace, idx + direction, idx)


_L = 16


def _make_sc_kernel(R, T, rows_per_worker, RB):
    mesh = plsc.VectorSubcoreMesh(core_axis_name="c", subcore_axis_name="s")
    n_blocks = rows_per_worker // RB
    chunks = T // _L

    @functools.partial(
        pl.kernel,
        mesh=mesh,
        out_type=jax.ShapeDtypeStruct((R * T,), jnp.float32),
        compiler_params=pltpu.CompilerParams(needs_layout_passes=False),
        scratch_types=[
            pltpu.VMEM((T,), jnp.int32),
            pltpu.VMEM((RB * T,), jnp.float32),
            pltpu.VMEM((RB * T,), jnp.float32),
        ],
    )
    def k(x_hbm, nb_hbm, out_hbm, idx_v, in_v, out_v):
        wid = lax.axis_index("s") * 2 + lax.axis_index("c")
        pltpu.sync_copy(nb_hbm, idx_v)
        row0 = wid * rows_per_worker

        def block(b, _):
            base = (row0 + b * RB) * T
            pltpu.sync_copy(x_hbm.at[pl.ds(base, RB * T)], in_v)

            def chunk(c, _):
                nb = idx_v[pl.ds(c * _L, _L)]
                for r in range(RB):
                    v = plsc.load_gather(in_v, [nb + r * T])
                    out_v[pl.ds(c * _L + r * T, _L)] = v
                return 0

            lax.fori_loop(0, chunks, chunk, 0)
            pltpu.sync_copy(out_v, out_hbm.at[pl.ds(base, RB * T)])
            return 0

        lax.fori_loop(0, n_blocks, block, 0)

    return k


def kernel(quantized):
    B, C, T = quantized.shape
    R = B * C
    n_workers = 32
    rows_per_worker = R // n_workers
    RB = 8
    neighbor = _neighbor_indices(T).astype(jnp.int32)
    x = quantized.reshape(R * T)
    out = _make_sc_kernel(R, T, rows_per_worker, RB)(x, neighbor)
    return out.reshape(B, C, T)


# speedup vs baseline: 1.6531x; 1.6531x over previous
import functools

import jax
import jax.numpy as jnp
from jax import lax
from jax.experimental import pallas as pl
from jax.experimental.pallas import tpu as pltpu
from jax.experimental.pallas import tpu_sc as plsc

_PROB = 0.12


def _neighbor_indices(T):
    k1, k2 = jax.random.split(jax.random.key(42))
    replace = jax.random.bernoulli(k1, _PROB, (T,))
    direction = jnp.where(jax.random.bernoulli(k2, 0.5, (T,)), 1, -1)
    idx = jnp.arange(T)
    direction = jnp.where(idx == 0, 1, direction)
    direction = jnp.where(idx == T - 1, -1, direction)
    return jnp.where(replace, idx + direction, idx)


_L = 16


def _make_sc_kernel(R, T, rows_per_worker, RB):
    mesh = plsc.VectorSubcoreMesh(core_axis_name="c", subcore_axis_name="s")
    n_blocks = rows_per_worker // RB
    chunks = T // _L

    @functools.partial(
        pl.kernel,
        mesh=mesh,
        out_type=jax.ShapeDtypeStruct((R * T,), jnp.float32),
        compiler_params=pltpu.CompilerParams(needs_layout_passes=False),
        scratch_types=[
            pltpu.VMEM((T,), jnp.int32),
            pltpu.VMEM((RB * T,), jnp.float32),
            pltpu.VMEM((RB * T,), jnp.float32),
        ],
    )
    def k(x_hbm, nb_hbm, out_hbm, idx_v, in_v, out_v):
        wid = lax.axis_index("s") * 2 + lax.axis_index("c")
        pltpu.sync_copy(nb_hbm, idx_v)
        row0 = wid * rows_per_worker

        def block(b, _):
            base = (row0 + b * RB) * T
            pltpu.sync_copy(x_hbm.at[pl.ds(base, RB * T)], in_v)

            @plsc.parallel_loop(0, T, _L, unroll=2)
            def chunk(t0):
                nb = idx_v[pl.ds(t0, _L)]
                for r in range(RB):
                    v = plsc.load_gather(in_v, [nb + r * T])
                    out_v[pl.ds(t0 + r * T, _L)] = v

            pltpu.sync_copy(out_v, out_hbm.at[pl.ds(base, RB * T)])
            return 0

        lax.fori_loop(0, n_blocks, block, 0)

    return k


def kernel(quantized):
    B, C, T = quantized.shape
    R = B * C
    n_workers = 32
    rows_per_worker = R // n_workers
    RB = 8
    neighbor = _neighbor_indices(T).astype(jnp.int32)
    x = quantized.reshape(R * T)
    out = _make_sc_kernel(R, T, rows_per_worker, RB)(x, neighbor)
    return out.reshape(B, C, T)


# speedup vs baseline: 1.9352x; 1.1706x over previous
import functools

import jax
import jax.numpy as jnp
from jax import lax
from jax.experimental import pallas as pl
from jax.experimental.pallas import tpu as pltpu
from jax.experimental.pallas import tpu_sc as plsc

_PROB = 0.12


def _neighbor_indices(T):
    k1, k2 = jax.random.split(jax.random.key(42))
    replace = jax.random.bernoulli(k1, _PROB, (T,))
    direction = jnp.where(jax.random.bernoulli(k2, 0.5, (T,)), 1, -1)
    idx = jnp.arange(T)
    direction = jnp.where(idx == 0, 1, direction)
    direction = jnp.where(idx == T - 1, -1, direction)
    return jnp.where(replace, idx + direction, idx)


_L = 16


def _make_sc_kernel(R, T, rows_per_worker, RB):
    mesh = plsc.VectorSubcoreMesh(core_axis_name="c", subcore_axis_name="s")
    n_blocks = rows_per_worker // RB
    blk = RB * T

    @functools.partial(
        pl.kernel,
        mesh=mesh,
        out_type=jax.ShapeDtypeStruct((R * T,), jnp.float32),
        compiler_params=pltpu.CompilerParams(needs_layout_passes=False),
        scratch_types=[
            pltpu.VMEM((T,), jnp.int32),
            pltpu.VMEM((blk,), jnp.float32),
            pltpu.VMEM((blk,), jnp.float32),
            pltpu.VMEM((blk,), jnp.float32),
            pltpu.VMEM((blk,), jnp.float32),
            pltpu.SemaphoreType.DMA,
            pltpu.SemaphoreType.DMA,
            pltpu.SemaphoreType.DMA,
            pltpu.SemaphoreType.DMA,
        ],
    )
    def k(x_hbm, nb_hbm, out_hbm, idx_v, in0, in1, out0, out1,
          sin0, sin1, sout0, sout1):
        wid = lax.axis_index("s") * 2 + lax.axis_index("c")
        pltpu.sync_copy(nb_hbm, idx_v)
        base0 = wid * rows_per_worker * T
        ins = (in0, in1)
        outs = (out0, out1)
        sins = (sin0, sin1)
        souts = (sout0, sout1)

        def start_in(b, ph):
            pltpu.async_copy(x_hbm.at[pl.ds(base0 + b * blk, blk)],
                             ins[ph], sins[ph])

        def wait_in(ph):
            pltpu.make_async_copy(x_hbm.at[pl.ds(base0, blk)],
                                  ins[ph], sins[ph]).wait()

        def start_out(b, ph):
            pltpu.async_copy(outs[ph],
                             out_hbm.at[pl.ds(base0 + b * blk, blk)],
                             souts[ph])

        def wait_out(ph):
            pltpu.make_async_copy(outs[ph],
                                  out_hbm.at[pl.ds(base0, blk)],
                                  souts[ph]).wait()

        start_in(0, 0)
        start_in(1, 1)

        @pl.loop(0, n_blocks, step=2)
        def blocks(g):
            for ph in (0, 1):
                b = g + ph
                wait_in(ph)

                @pl.when(b >= 2)
                def _():
                    wait_out(ph)

                in_v = ins[ph]
                out_v = outs[ph]

                @plsc.parallel_loop(0, T, _L, unroll=4)
                def chunk(t0):
                    nb = idx_v[pl.ds(t0, _L)]
                    for r in range(RB):
                        v = plsc.load_gather(in_v, [nb + r * T])
                        out_v[pl.ds(t0 + r * T, _L)] = v

                start_out(b, ph)

                @pl.when(b + 2 < n_blocks)
                def _():
                    start_in(b + 2, ph)

        wait_out(0)
        wait_out(1)

    return k


def kernel(quantized):
    B, C, T = quantized.shape
    R = B * C
    n_workers = 32
    rows_per_worker = R // n_workers
    RB = 4
    neighbor = _neighbor_indices(T).astype(jnp.int32)
    x = quantized.reshape(R * T)
    out = _make_sc_kernel(R, T, rows_per_worker, RB)(x, neighbor)
    return out.reshape(B, C, T)
